# baseline (speedup 1.0000x reference)
import functools

import jax
import jax.numpy as jnp
from jax import lax
from jax.experimental import pallas as pl
from jax.experimental.pallas import tpu as pltpu
from jax.experimental.pallas import tpu_sc as plsc

NC = 2
NS = 16
NW = NC * NS
LANES = 16
ACC_W = 6
NP = 10112
PART = ACC_W * NP
PSTRIDE = 61440


def _proj_body(N, x_ref, w1_ref, w2_ref, b_ref, o_ref):
    xb = x_ref[...]
    o_ref[pl.ds(0, N), :] = (
        jnp.dot(xb, w1_ref[...], preferred_element_type=jnp.float32) + b_ref[...]
    )
    o_ref[pl.ds(N, N), :] = jnp.dot(
        xb, w2_ref[...], preferred_element_type=jnp.float32
    )


def _sum_body(p_ref, o_ref):
    h = (
        (p_ref[pl.ds(0, PSTRIDE)] + p_ref[pl.ds(PSTRIDE, PSTRIDE)])
        + (p_ref[pl.ds(2 * PSTRIDE, PSTRIDE)] + p_ref[pl.ds(3 * PSTRIDE, PSTRIDE)])
    )

    @pl.when(pl.program_id(0) == 0)
    def _():
        o_ref[...] = h

    @pl.when(pl.program_id(0) != 0)
    def _():
        o_ref[...] += h


def _final_body(N, x_ref, nb_ref, anc_ref, o_ref):
    nb = nb_ref[...][:, :N]
    deg = nb[5:6, :]
    scale = 1.0 / jnp.maximum(deg, 1.0)
    nbs = nb * scale
    o_ref[...] = x_ref[...] + lax.dot_general(
        nbs, anc_ref[...], (((0,), (0,)), ((), ())),
        preferred_element_type=jnp.float32,
    )


def _sc_edge_body(A, EW, BE, t_hbm, srcarr_hbm, dstoff_hbm, z_hbm, out_hbm,
                  src_v, dst0, dst1, gs0, gs1, gd0, gd1, acc_v,
                  sem_s0, sem_s1, sem_d0, sem_d1):
    c = lax.axis_index("c")
    s = lax.axis_index("s")
    wid = s * NC + c
    base = wid * EW
    nbatch = EW // BE
    nchunk = BE // LANES

    cols = [jnp.full((LANES,), a, jnp.int32) for a in range(A)]
    iota = lax.iota(jnp.int32, LANES)
    ones = jnp.ones((LANES,), jnp.float32)

    dsts = (dst0, dst1)
    gss, gds = (gs0, gs1), (gd0, gd1)
    sem_ss, sem_ds = (sem_s0, sem_s1), (sem_d0, sem_d1)

    def load_batch(j):
        b = j % 2
        pltpu.sync_copy(srcarr_hbm.at[pl.ds(base + j * BE, BE)], src_v)
        pltpu.sync_copy(dstoff_hbm.at[pl.ds(base + j * BE, BE)], dsts[b])
        ds_ = pltpu.async_copy(t_hbm.at[src_v], gss[b], sem_ss[b])
        dd_ = pltpu.async_copy(t_hbm.at[dsts[b]], gds[b], sem_ds[b])
        return ds_, dd_

    pending = load_batch(0)
    pltpu.sync_copy(z_hbm, acc_v)

    for j in range(nbatch):
        b = j % 2
        pending[0].wait()
        pending[1].wait()
        if j + 1 < nbatch:
            nxt = load_batch(j + 1)
        dst_v, gsrc_v, gdst_v = dsts[b], gss[b], gds[b]

        def chunk(k, carry):
            eoff = k * LANES + iota
            didx = dst_v[pl.ds(k * LANES, LANES)]
            logit = [
                plsc.load_gather(gsrc_v, [eoff, cols[a]])
                + plsc.load_gather(gdst_v, [eoff, cols[a]])
                for a in range(A)
            ]
            logit = [jnp.where(v >= 0.0, v, v * 0.01) for v in logit]
            ex = [jnp.exp(v) for v in logit]
            tot = ex[0]
            for v in ex[1:]:
                tot = tot + v
            inv = 1.0 / tot
            n_nodes = t_hbm.shape[0] // 2
            for a in range(A):
                plsc.addupdate_scatter(
                    acc_v, [didx + (a * NP - n_nodes)], ex[a] * inv
                )
            plsc.addupdate_scatter(acc_v, [didx + (A * NP - n_nodes)], ones)
            return carry

        lax.fori_loop(0, nchunk, chunk, 0)
        if j + 1 < nbatch:
            pending = nxt

    pltpu.sync_copy(acc_v, out_hbm.at[pl.ds(wid * PSTRIDE, PART)])


def kernel(x, edge_index, anchor_prompt, W, b_w):
    N, D = x.shape
    E = edge_index.shape[1]
    A = anchor_prompt.shape[0]
    assert E % NW == 0 and N <= NP
    EW = E // NW
    BE = 2000
    assert EW % BE == 0 and BE % LANES == 0

    W1t = W[:, :D].T
    W2t = W[:, D:].T
    bvec = b_w.reshape(1, A)
    anc_pad = jnp.zeros((ACC_W, D), jnp.float32).at[:A].set(anchor_prompt)
    zbuf = jnp.zeros((PART,), jnp.float32)

    Tcat = pl.pallas_call(
        functools.partial(_proj_body, N),
        out_shape=jax.ShapeDtypeStruct((2 * N, A), jnp.float32),
    )(x, W1t, W2t, bvec)
    srcarr = edge_index[0]
    dstoff = edge_index[1] + N

    mesh = plsc.VectorSubcoreMesh(
        core_axis_name="c", subcore_axis_name="s",
        num_cores=NC, num_subcores=NS,
    )
    sc_fn = pl.kernel(
        functools.partial(_sc_edge_body, A, EW, BE),
        out_type=jax.ShapeDtypeStruct((NW * PSTRIDE,), jnp.float32),
        mesh=mesh,
        compiler_params=pltpu.CompilerParams(
            use_tc_tiling_on_sc=False, needs_layout_passes=False
        ),
        scratch_types=[
            pltpu.VMEM((BE,), jnp.int32),
            pltpu.VMEM((BE,), jnp.int32),
            pltpu.VMEM((BE,), jnp.int32),
            pltpu.VMEM((BE, A), jnp.float32),
            pltpu.VMEM((BE, A), jnp.float32),
            pltpu.VMEM((BE, A), jnp.float32),
            pltpu.VMEM((BE, A), jnp.float32),
            pltpu.VMEM((PART,), jnp.float32),
            pltpu.SemaphoreType.DMA,
            pltpu.SemaphoreType.DMA,
            pltpu.SemaphoreType.DMA,
            pltpu.SemaphoreType.DMA,
        ],
    )
    parts = sc_fn(Tcat, srcarr, dstoff, zbuf)

    nb_flat = pl.pallas_call(
        _sum_body,
        grid=(NW // 4,),
        in_specs=[pl.BlockSpec((4 * PSTRIDE,), lambda i: (i,))],
        out_specs=pl.BlockSpec((PSTRIDE,), lambda i: (0,)),
        out_shape=jax.ShapeDtypeStruct((PSTRIDE,), jnp.float32),
    )(parts)
    nb6 = nb_flat[:PART].reshape(ACC_W, NP)

    out = pl.pallas_call(
        functools.partial(_final_body, N),
        out_shape=jax.ShapeDtypeStruct((N, D), jnp.float32),
    )(x, nb6, anc_pad)
    return out


# speedup vs baseline: 1.0946x; 1.0946x over previous
import functools

import jax
import jax.numpy as jnp
from jax import lax
from jax.experimental import pallas as pl
from jax.experimental.pallas import tpu as pltpu
from jax.experimental.pallas import tpu_sc as plsc

NC = 2
NS = 16
NW = NC * NS
LANES = 16
ACC_W = 6
NP = 10112
PART = ACC_W * NP
PSTRIDE = 61440


def _proj_body(x_ref, w1_ref, w2_ref, b_ref, o1_ref, o2_ref):
    xb = x_ref[...]
    o1_ref[...] = (
        jnp.dot(xb, w1_ref[...], preferred_element_type=jnp.float32) + b_ref[...]
    )
    o2_ref[...] = jnp.dot(xb, w2_ref[...], preferred_element_type=jnp.float32)


def _sum_body(p_ref, o_ref):
    h = (
        (p_ref[pl.ds(0, PSTRIDE)] + p_ref[pl.ds(PSTRIDE, PSTRIDE)])
        + (p_ref[pl.ds(2 * PSTRIDE, PSTRIDE)] + p_ref[pl.ds(3 * PSTRIDE, PSTRIDE)])
    )

    @pl.when(pl.program_id(0) == 0)
    def _():
        o_ref[...] = h

    @pl.when(pl.program_id(0) != 0)
    def _():
        o_ref[...] += h


def _final_body(N, x_ref, nb_ref, anc_ref, o_ref):
    nb = nb_ref[...][:, :N]
    deg = nb[5:6, :]
    scale = 1.0 / jnp.maximum(deg, 1.0)
    nbs = nb * scale
    o_ref[...] = x_ref[...] + lax.dot_general(
        nbs, anc_ref[...], (((0,), (0,)), ((), ())),
        preferred_element_type=jnp.float32,
    )


def _sc_edge_body(A, EW, BE, t1_hbm, t2_hbm, ei_hbm, z_hbm, out_hbm,
                  src_v, dst0, dst1, gs0, gs1, gd0, gd1, acc_v,
                  sem_s0, sem_s1, sem_d0, sem_d1):
    c = lax.axis_index("c")
    s = lax.axis_index("s")
    wid = s * NC + c
    base = wid * EW
    nbatch = EW // BE
    nchunk = BE // LANES

    cols = [jnp.full((LANES,), a, jnp.int32) for a in range(A)]
    iota = lax.iota(jnp.int32, LANES)
    ones = jnp.ones((LANES,), jnp.float32)

    dsts = (dst0, dst1)
    gss, gds = (gs0, gs1), (gd0, gd1)
    sem_ss, sem_ds = (sem_s0, sem_s1), (sem_d0, sem_d1)

    def load_batch(j):
        b = j % 2
        pltpu.sync_copy(ei_hbm.at[0, pl.ds(base + j * BE, BE)], src_v)
        pltpu.sync_copy(ei_hbm.at[1, pl.ds(base + j * BE, BE)], dsts[b])
        ds_ = pltpu.async_copy(t1_hbm.at[src_v], gss[b], sem_ss[b])
        dd_ = pltpu.async_copy(t2_hbm.at[dsts[b]], gds[b], sem_ds[b])
        return ds_, dd_

    pending = load_batch(0)
    pltpu.sync_copy(z_hbm, acc_v)

    for j in range(nbatch):
        b = j % 2
        pending[0].wait()
        pending[1].wait()
        if j + 1 < nbatch:
            nxt = load_batch(j + 1)
        dst_v, gsrc_v, gdst_v = dsts[b], gss[b], gds[b]

        def chunk(k, carry):
            eoff = k * LANES + iota
            didx = dst_v[pl.ds(k * LANES, LANES)]
            logit = [
                plsc.load_gather(gsrc_v, [eoff, cols[a]])
                + plsc.load_gather(gdst_v, [eoff, cols[a]])
                for a in range(A)
            ]
            logit = [jnp.where(v >= 0.0, v, v * 0.01) for v in logit]
            ex = [jnp.exp(v) for v in logit]
            tot = ex[0]
            for v in ex[1:]:
                tot = tot + v
            inv = 1.0 / tot
            for a in range(A):
                plsc.addupdate_scatter(acc_v, [didx + a * NP], ex[a] * inv)
            plsc.addupdate_scatter(acc_v, [didx + A * NP], ones)
            return carry

        lax.fori_loop(0, nchunk, chunk, 0)
        if j + 1 < nbatch:
            pending = nxt

    pltpu.sync_copy(acc_v, out_hbm.at[pl.ds(wid * PSTRIDE, PART)])


def kernel(x, edge_index, anchor_prompt, W, b_w):
    N, D = x.shape
    E = edge_index.shape[1]
    A = anchor_prompt.shape[0]
    assert E % NW == 0 and N <= NP
    EW = E // NW
    BE = 2000
    assert EW % BE == 0 and BE % LANES == 0

    W1t = W[:, :D].T
    W2t = W[:, D:].T
    bvec = b_w.reshape(1, A)
    anc_pad = jnp.zeros((ACC_W, D), jnp.float32).at[:A].set(anchor_prompt)
    zbuf = jnp.zeros((PART,), jnp.float32)

    T1, T2 = pl.pallas_call(
        _proj_body,
        out_shape=[
            jax.ShapeDtypeStruct((N, A), jnp.float32),
            jax.ShapeDtypeStruct((N, A), jnp.float32),
        ],
    )(x, W1t, W2t, bvec)

    mesh = plsc.VectorSubcoreMesh(
        core_axis_name="c", subcore_axis_name="s",
        num_cores=NC, num_subcores=NS,
    )
    sc_fn = pl.kernel(
        functools.partial(_sc_edge_body, A, EW, BE),
        out_type=jax.ShapeDtypeStruct((NW * PSTRIDE,), jnp.float32),
        mesh=mesh,
        compiler_params=pltpu.CompilerParams(
            use_tc_tiling_on_sc=False, needs_layout_passes=False
        ),
        scratch_types=[
            pltpu.VMEM((BE,), jnp.int32),
            pltpu.VMEM((BE,), jnp.int32),
            pltpu.VMEM((BE,), jnp.int32),
            pltpu.VMEM((BE, A), jnp.float32),
            pltpu.VMEM((BE, A), jnp.float32),
            pltpu.VMEM((BE, A), jnp.float32),
            pltpu.VMEM((BE, A), jnp.float32),
            pltpu.VMEM((PART,), jnp.float32),
            pltpu.SemaphoreType.DMA,
            pltpu.SemaphoreType.DMA,
            pltpu.SemaphoreType.DMA,
            pltpu.SemaphoreType.DMA,
        ],
    )
    parts = sc_fn(T1, T2, edge_index, zbuf)

    nb_flat = pl.pallas_call(
        _sum_body,
        grid=(NW // 4,),
        in_specs=[pl.BlockSpec((4 * PSTRIDE,), lambda i: (i,))],
        out_specs=pl.BlockSpec((PSTRIDE,), lambda i: (0,)),
        out_shape=jax.ShapeDtypeStruct((PSTRIDE,), jnp.float32),
    )(parts)
    nb6 = nb_flat[:PART].reshape(ACC_W, NP)

    out = pl.pallas_call(
        functools.partial(_final_body, N),
        out_shape=jax.ShapeDtypeStruct((N, D), jnp.float32),
    )(x, nb6, anc_pad)
    return out
